# Initial kernel scaffold; baseline (speedup 1.0000x reference)
import jax
import jax.numpy as jnp
from jax.experimental import pallas as pl


def kernel(logits, targets):
    raise NotImplementedError("write your pallas kernel here")


# speedup vs baseline: 13.9001x; 13.9001x over previous
import functools

import jax
import jax.numpy as jnp
from jax import lax
from jax.experimental import pallas as pl
from jax.experimental.pallas import tpu as pltpu
from jax.experimental.pallas import tpu_sc as plsc

N = 16 * 512 * 512
K = N // 4
ROWS, COLS = 4096, 1024
GRID_R = 8
BLK_R = ROWS // GRID_R
NW = 32
CHUNK = N // NW
SUB = 8192
NSUB = CHUNK // SUB
BINS = 4096
SH1 = 19
SH2 = 7
LANES = 16


def _bce_body(x_ref, t_ref, o_ref):
    x = x_ref[...]
    t = t_ref[...]
    loss = jnp.maximum(x, 0.0) - x * t + jnp.log1p(jnp.exp(-jnp.abs(x)))
    o_ref[...] = lax.bitcast_convert_type(loss, jnp.int32)


_sc_mesh = plsc.VectorSubcoreMesh(core_axis_name="c", subcore_axis_name="s")


@functools.partial(
    pl.kernel,
    out_type=jax.ShapeDtypeStruct((NW, BINS), jnp.int32),
    mesh=_sc_mesh,
    compiler_params=pltpu.CompilerParams(needs_layout_passes=False),
    scratch_types=[
        pltpu.VMEM((SUB,), jnp.int32),
        pltpu.VMEM((BINS * LANES,), jnp.int32),
        pltpu.VMEM((BINS,), jnp.int32),
    ],
)
def _hist1_sc(loss_hbm, out_hbm, buf, hist, obuf):
    wid = lax.axis_index("s") * 2 + lax.axis_index("c")
    base = wid * CHUNK
    lanes = lax.iota(jnp.int32, LANES)
    zeros16 = jnp.zeros((LANES,), jnp.int32)
    ones16 = jnp.ones((LANES,), jnp.int32)

    def zero_body(i, c):
        hist[pl.ds(i * LANES, LANES)] = zeros16
        return c

    lax.fori_loop(0, BINS, zero_body, 0)

    for s in range(NSUB):
        pltpu.sync_copy(loss_hbm.at[pl.ds(base + s * SUB, SUB)], buf)

        def body(i, c):
            u = buf[pl.ds(i * LANES, LANES)]
            d = u >> SH1
            plsc.addupdate_scatter(hist, [d * LANES + lanes], ones16)
            return c

        lax.fori_loop(0, SUB // LANES, body, 0)

    def red_body(g, c):
        rows = g * LANES + lanes
        acc = zeros16
        for l in range(LANES):
            acc = acc + plsc.load_gather(hist, [rows * LANES + l])
        obuf[pl.ds(g * LANES, LANES)] = acc
        return c

    lax.fori_loop(0, BINS // LANES, red_body, 0)
    pltpu.sync_copy(obuf, out_hbm.at[wid])


@functools.partial(
    pl.kernel,
    out_type=jax.ShapeDtypeStruct((NW, BINS), jnp.int32),
    mesh=_sc_mesh,
    compiler_params=pltpu.CompilerParams(needs_layout_passes=False),
    scratch_types=[
        pltpu.VMEM((SUB,), jnp.int32),
        pltpu.VMEM((BINS * LANES,), jnp.int32),
        pltpu.VMEM((BINS,), jnp.int32),
        pltpu.VMEM((LANES,), jnp.int32),
    ],
)
def _hist2_sc(loss_hbm, b1_hbm, out_hbm, buf, hist, obuf, b1buf):
    wid = lax.axis_index("s") * 2 + lax.axis_index("c")
    base = wid * CHUNK
    lanes = lax.iota(jnp.int32, LANES)
    zeros16 = jnp.zeros((LANES,), jnp.int32)
    ones16 = jnp.ones((LANES,), jnp.int32)

    pltpu.sync_copy(b1_hbm, b1buf)
    b1v = b1buf[...]

    def zero_body(i, c):
        hist[pl.ds(i * LANES, LANES)] = zeros16
        return c

    lax.fori_loop(0, BINS, zero_body, 0)

    for s in range(NSUB):
        pltpu.sync_copy(loss_hbm.at[pl.ds(base + s * SUB, SUB)], buf)

        def body(i, c):
            u = buf[pl.ds(i * LANES, LANES)]
            d1 = u >> SH1
            d2 = (u >> SH2) & 0xFFF
            plsc.addupdate_scatter(
                hist, [d2 * LANES + lanes], ones16, mask=d1 == b1v)
            return c

        lax.fori_loop(0, SUB // LANES, body, 0)

    def red_body(g, c):
        rows = g * LANES + lanes
        acc = zeros16
        for l in range(LANES):
            acc = acc + plsc.load_gather(hist, [rows * LANES + l])
        obuf[pl.ds(g * LANES, LANES)] = acc
        return c

    lax.fori_loop(0, BINS // LANES, red_body, 0)
    pltpu.sync_copy(obuf, out_hbm.at[wid])


def _suffix_counts(h_ref):
    hw = h_ref[...].astype(jnp.float32)
    h = jnp.zeros((32, 128), jnp.float32)
    for w in range(NW):
        h = h + hw[w * 32:(w + 1) * 32, :]
    cs = lax.broadcasted_iota(jnp.int32, (128, 128), 0)
    ct = lax.broadcasted_iota(jnp.int32, (128, 128), 1)
    m_in = (cs >= ct).astype(jnp.float32)
    inrow = jnp.dot(h, m_in, preferred_element_type=jnp.float32)
    rs = lax.broadcasted_iota(jnp.int32, (32, 32), 0)
    rt = lax.broadcasted_iota(jnp.int32, (32, 32), 1)
    m_tail = (rt > rs).astype(jnp.float32)
    tail = jnp.sum(jnp.dot(m_tail, h, preferred_element_type=jnp.float32),
                   axis=1, keepdims=True)
    idx = (lax.broadcasted_iota(jnp.int32, (32, 128), 0) * 128
           + lax.broadcasted_iota(jnp.int32, (32, 128), 1))
    return inrow + tail, idx, h


def _pick1_body(h_ref, o_ref):
    C, idx, h = _suffix_counts(h_ref)
    b1 = jnp.max(jnp.where(C >= float(K), idx, -1))
    c_above = jnp.sum(jnp.where(idx > b1, h, 0.0))
    r1 = (float(K) - c_above).astype(jnp.int32)
    orow = lax.broadcasted_iota(jnp.int32, (8, 128), 0)
    ocol = lax.broadcasted_iota(jnp.int32, (8, 128), 1)
    flat = orow * 128 + ocol
    o_ref[...] = jnp.where(flat == 0, b1, jnp.where(flat == 1, r1, 0))


def _pick2_body(h_ref, c1_ref, o_ref):
    b1 = c1_ref[0, 0]
    r1 = c1_ref[0, 1].astype(jnp.float32)
    C, idx, _ = _suffix_counts(h_ref)
    b2 = jnp.max(jnp.where(C >= r1, idx, -1))
    edge = b1 * (1 << SH1) + b2 * (1 << SH2)
    orow = lax.broadcasted_iota(jnp.int32, (8, 128), 0)
    ocol = lax.broadcasted_iota(jnp.int32, (8, 128), 1)
    flat = orow * 128 + ocol
    o_ref[...] = jnp.where(flat == 0, edge, 0)


def _final_body(c2_ref, x_ref, o_ref):
    i = pl.program_id(0)
    edge = c2_ref[0, 0]
    u = x_ref[...]
    x = lax.bitcast_convert_type(u, jnp.float32)
    keep = u >= edge + (1 << SH2)
    s_part = jnp.sum(jnp.where(keep, x, 0.0))
    c_part = jnp.sum(keep.astype(jnp.float32))
    lane = lax.broadcasted_iota(jnp.int32, (1, 128), 1)
    part = jnp.where(lane == 0, s_part, jnp.where(lane == 1, c_part, 0.0))

    @pl.when(i == 0)
    def _():
        o_ref[...] = jnp.zeros_like(o_ref)

    o_ref[...] += part

    @pl.when(i == GRID_R - 1)
    def _():
        r = o_ref[...]
        s_tot = r[0, 0]
        c_tot = r[0, 1]
        edge_f = lax.bitcast_convert_type(edge, jnp.float32)
        res = (s_tot + (float(K) - c_tot) * edge_f) / float(K)
        o_ref[...] = jnp.where(lane == 2, res, r)


def kernel(logits, targets):
    x2d = logits.reshape(ROWS, COLS)
    t2d = targets.reshape(ROWS, COLS)

    loss = pl.pallas_call(
        _bce_body,
        grid=(GRID_R,),
        in_specs=[pl.BlockSpec((BLK_R, COLS), lambda i: (i, 0))] * 2,
        out_specs=pl.BlockSpec((BLK_R, COLS), lambda i: (i, 0)),
        out_shape=jax.ShapeDtypeStruct((ROWS, COLS), jnp.int32),
    )(x2d, t2d)

    loss_flat = loss.reshape(N)

    h1 = _hist1_sc(loss_flat)
    c1 = pl.pallas_call(
        _pick1_body,
        out_shape=jax.ShapeDtypeStruct((8, 128), jnp.int32),
    )(h1.reshape(1024, 128))

    b1_arr = jnp.full((LANES,), c1[0, 0], jnp.int32)
    h2 = _hist2_sc(loss_flat, b1_arr)
    c2 = pl.pallas_call(
        _pick2_body,
        out_shape=jax.ShapeDtypeStruct((8, 128), jnp.int32),
    )(h2.reshape(1024, 128), c1)

    outv = pl.pallas_call(
        _final_body,
        grid=(GRID_R,),
        in_specs=[
            pl.BlockSpec((8, 128), lambda i: (0, 0)),
            pl.BlockSpec((BLK_R, COLS), lambda i: (i, 0)),
        ],
        out_specs=pl.BlockSpec((1, 128), lambda i: (0, 0)),
        out_shape=jax.ShapeDtypeStruct((1, 128), jnp.float32),
    )(c2, loss)

    return outv[0, 2]


# speedup vs baseline: 34.7509x; 2.5001x over previous
import functools

import jax
import jax.numpy as jnp
from jax import lax
from jax.experimental import pallas as pl
from jax.experimental.pallas import tpu as pltpu
from jax.experimental.pallas import tpu_sc as plsc

N = 16 * 512 * 512
K = N // 4
ROWS, COLS = 4096, 1024
GRID_R = 8
BLK_R = ROWS // GRID_R
NW = 32
CHUNK = N // NW
SUB = 8192
NSUB = CHUNK // SUB
BINS = 4096
SH1 = 19
SH2 = 7
LANES = 16


def _bce_body(x_ref, t_ref, o_ref):
    x = x_ref[...]
    t = t_ref[...]
    loss = jnp.maximum(x, 0.0) - x * t + jnp.log1p(jnp.exp(-jnp.abs(x)))
    o_ref[...] = lax.bitcast_convert_type(loss, jnp.int32)


_sc_mesh = plsc.VectorSubcoreMesh(core_axis_name="c", subcore_axis_name="s")


@functools.partial(
    pl.kernel,
    out_type=jax.ShapeDtypeStruct((NW, BINS), jnp.int32),
    mesh=_sc_mesh,
    compiler_params=pltpu.CompilerParams(needs_layout_passes=False),
    scratch_types=[
        pltpu.VMEM((2, SUB), jnp.int32),
        pltpu.VMEM((BINS * LANES,), jnp.int32),
        pltpu.VMEM((BINS,), jnp.int32),
        pltpu.SemaphoreType.DMA((2,)),
    ],
)
def _hist1_sc(loss_hbm, out_hbm, bufs, hist, obuf, sems):
    wid = lax.axis_index("s") * 2 + lax.axis_index("c")
    base = wid * CHUNK
    lanes = lax.iota(jnp.int32, LANES)
    zeros16 = jnp.zeros((LANES,), jnp.int32)
    ones16 = jnp.ones((LANES,), jnp.int32)

    @functools.partial(plsc.parallel_loop, 0, BINS, unroll=8)
    def _(i):
        hist[pl.ds(i * LANES, LANES)] = zeros16

    cps = [None, None]
    cps[0] = pltpu.async_copy(
        loss_hbm.at[pl.ds(base, SUB)], bufs.at[0], sems.at[0])
    for s in range(NSUB):
        cur = s % 2
        if s + 1 < NSUB:
            nxt = (s + 1) % 2
            cps[nxt] = pltpu.async_copy(
                loss_hbm.at[pl.ds(base + (s + 1) * SUB, SUB)],
                bufs.at[nxt], sems.at[nxt])
        cps[cur].wait()
        buf = bufs.at[cur]

        @functools.partial(plsc.parallel_loop, 0, SUB // LANES, unroll=8)
        def _(i):
            u = buf[pl.ds(i * LANES, LANES)]
            d = u >> SH1
            plsc.addupdate_scatter(hist, [d * LANES + lanes], ones16)

    @functools.partial(plsc.parallel_loop, 0, BINS // LANES, unroll=4)
    def _(g):
        rows = g * LANES + lanes
        acc = zeros16
        for l in range(LANES):
            acc = acc + plsc.load_gather(hist, [rows * LANES + l])
        obuf[pl.ds(g * LANES, LANES)] = acc

    pltpu.sync_copy(obuf, out_hbm.at[wid])


@functools.partial(
    pl.kernel,
    out_type=jax.ShapeDtypeStruct((NW, BINS), jnp.int32),
    mesh=_sc_mesh,
    compiler_params=pltpu.CompilerParams(needs_layout_passes=False),
    scratch_types=[
        pltpu.VMEM((2, SUB), jnp.int32),
        pltpu.VMEM((BINS * LANES,), jnp.int32),
        pltpu.VMEM((BINS,), jnp.int32),
        pltpu.VMEM((LANES,), jnp.int32),
        pltpu.SemaphoreType.DMA((2,)),
    ],
)
def _hist2_sc(loss_hbm, b1_hbm, out_hbm, bufs, hist, obuf, b1buf, sems):
    wid = lax.axis_index("s") * 2 + lax.axis_index("c")
    base = wid * CHUNK
    lanes = lax.iota(jnp.int32, LANES)
    zeros16 = jnp.zeros((LANES,), jnp.int32)
    ones16 = jnp.ones((LANES,), jnp.int32)

    pltpu.sync_copy(b1_hbm, b1buf)
    b1v = b1buf[...]

    @functools.partial(plsc.parallel_loop, 0, BINS, unroll=8)
    def _(i):
        hist[pl.ds(i * LANES, LANES)] = zeros16

    cps = [None, None]
    cps[0] = pltpu.async_copy(
        loss_hbm.at[pl.ds(base, SUB)], bufs.at[0], sems.at[0])
    for s in range(NSUB):
        cur = s % 2
        if s + 1 < NSUB:
            nxt = (s + 1) % 2
            cps[nxt] = pltpu.async_copy(
                loss_hbm.at[pl.ds(base + (s + 1) * SUB, SUB)],
                bufs.at[nxt], sems.at[nxt])
        cps[cur].wait()
        buf = bufs.at[cur]

        @functools.partial(plsc.parallel_loop, 0, SUB // LANES, unroll=8)
        def _(i):
            u = buf[pl.ds(i * LANES, LANES)]
            d1 = u >> SH1
            d2 = (u >> SH2) & 0xFFF
            plsc.addupdate_scatter(
                hist, [d2 * LANES + lanes], ones16, mask=d1 == b1v)

    @functools.partial(plsc.parallel_loop, 0, BINS // LANES, unroll=4)
    def _(g):
        rows = g * LANES + lanes
        acc = zeros16
        for l in range(LANES):
            acc = acc + plsc.load_gather(hist, [rows * LANES + l])
        obuf[pl.ds(g * LANES, LANES)] = acc

    pltpu.sync_copy(obuf, out_hbm.at[wid])


def _suffix_counts(h_ref):
    hw = h_ref[...].astype(jnp.float32)
    h = jnp.zeros((32, 128), jnp.float32)
    for w in range(NW):
        h = h + hw[w * 32:(w + 1) * 32, :]
    cs = lax.broadcasted_iota(jnp.int32, (128, 128), 0)
    ct = lax.broadcasted_iota(jnp.int32, (128, 128), 1)
    m_in = (cs >= ct).astype(jnp.float32)
    inrow = jnp.dot(h, m_in, preferred_element_type=jnp.float32)
    rs = lax.broadcasted_iota(jnp.int32, (32, 32), 0)
    rt = lax.broadcasted_iota(jnp.int32, (32, 32), 1)
    m_tail = (rt > rs).astype(jnp.float32)
    tail = jnp.sum(jnp.dot(m_tail, h, preferred_element_type=jnp.float32),
                   axis=1, keepdims=True)
    idx = (lax.broadcasted_iota(jnp.int32, (32, 128), 0) * 128
           + lax.broadcasted_iota(jnp.int32, (32, 128), 1))
    return inrow + tail, idx, h


def _pick1_body(h_ref, o_ref):
    C, idx, h = _suffix_counts(h_ref)
    b1 = jnp.max(jnp.where(C >= float(K), idx, -1))
    c_above = jnp.sum(jnp.where(idx > b1, h, 0.0))
    r1 = (float(K) - c_above).astype(jnp.int32)
    orow = lax.broadcasted_iota(jnp.int32, (8, 128), 0)
    ocol = lax.broadcasted_iota(jnp.int32, (8, 128), 1)
    flat = orow * 128 + ocol
    o_ref[...] = jnp.where(flat == 0, b1, jnp.where(flat == 1, r1, 0))


def _pick2_body(h_ref, c1_ref, o_ref):
    b1 = c1_ref[0, 0]
    r1 = c1_ref[0, 1].astype(jnp.float32)
    C, idx, _ = _suffix_counts(h_ref)
    b2 = jnp.max(jnp.where(C >= r1, idx, -1))
    edge = b1 * (1 << SH1) + b2 * (1 << SH2)
    orow = lax.broadcasted_iota(jnp.int32, (8, 128), 0)
    ocol = lax.broadcasted_iota(jnp.int32, (8, 128), 1)
    flat = orow * 128 + ocol
    o_ref[...] = jnp.where(flat == 0, edge, 0)


def _final_body(c2_ref, x_ref, o_ref):
    i = pl.program_id(0)
    edge = c2_ref[0, 0]
    u = x_ref[...]
    x = lax.bitcast_convert_type(u, jnp.float32)
    keep = u >= edge + (1 << SH2)
    s_part = jnp.sum(jnp.where(keep, x, 0.0))
    c_part = jnp.sum(keep.astype(jnp.float32))
    lane = lax.broadcasted_iota(jnp.int32, (1, 128), 1)
    part = jnp.where(lane == 0, s_part, jnp.where(lane == 1, c_part, 0.0))

    @pl.when(i == 0)
    def _():
        o_ref[...] = jnp.zeros_like(o_ref)

    o_ref[...] += part

    @pl.when(i == GRID_R - 1)
    def _():
        r = o_ref[...]
        s_tot = r[0, 0]
        c_tot = r[0, 1]
        edge_f = lax.bitcast_convert_type(edge, jnp.float32)
        res = (s_tot + (float(K) - c_tot) * edge_f) / float(K)
        o_ref[...] = jnp.where(lane == 2, res, r)


def kernel(logits, targets):
    x2d = logits.reshape(ROWS, COLS)
    t2d = targets.reshape(ROWS, COLS)

    loss = pl.pallas_call(
        _bce_body,
        grid=(GRID_R,),
        in_specs=[pl.BlockSpec((BLK_R, COLS), lambda i: (i, 0))] * 2,
        out_specs=pl.BlockSpec((BLK_R, COLS), lambda i: (i, 0)),
        out_shape=jax.ShapeDtypeStruct((ROWS, COLS), jnp.int32),
    )(x2d, t2d)

    loss_flat = loss.reshape(N)

    h1 = _hist1_sc(loss_flat)
    c1 = pl.pallas_call(
        _pick1_body,
        out_shape=jax.ShapeDtypeStruct((8, 128), jnp.int32),
    )(h1.reshape(1024, 128))

    b1_arr = jnp.full((LANES,), c1[0, 0], jnp.int32)
    h2 = _hist2_sc(loss_flat, b1_arr)
    c2 = pl.pallas_call(
        _pick2_body,
        out_shape=jax.ShapeDtypeStruct((8, 128), jnp.int32),
    )(h2.reshape(1024, 128), c1)

    outv = pl.pallas_call(
        _final_body,
        grid=(GRID_R,),
        in_specs=[
            pl.BlockSpec((8, 128), lambda i: (0, 0)),
            pl.BlockSpec((BLK_R, COLS), lambda i: (i, 0)),
        ],
        out_specs=pl.BlockSpec((1, 128), lambda i: (0, 0)),
        out_shape=jax.ShapeDtypeStruct((1, 128), jnp.float32),
    )(c2, loss)

    return outv[0, 2]


# speedup vs baseline: 63.4331x; 1.8254x over previous
import functools

import jax
import jax.numpy as jnp
from jax import lax
from jax.experimental import pallas as pl
from jax.experimental.pallas import tpu as pltpu
from jax.experimental.pallas import tpu_sc as plsc

N = 16 * 512 * 512
K = N // 4
ROWS, COLS = 8192, 512
GRID_R = 8
BLK_R = ROWS // GRID_R
NW = 32
TROWS = ROWS // NW
SUBR = 16
NSUB = TROWS // SUBR
SUB_V = SUBR * COLS // 16
BINS = 4096
HOUT = BINS // 128
SH1 = 19
SH2 = 7
LANES = 16


def _bce_body(x_ref, t_ref, o_ref):
    x = x_ref[...]
    t = t_ref[...]
    loss = jnp.maximum(x, 0.0) - x * t + jnp.log1p(jnp.exp(-jnp.abs(x)))
    o_ref[...] = lax.bitcast_convert_type(loss, jnp.int32)


_sc_mesh = plsc.VectorSubcoreMesh(core_axis_name="c", subcore_axis_name="s")


def _hist_tile(loss_hbm, bufs, hist, obuf, sems, base_row, scatter):
    lanes = lax.iota(jnp.int32, LANES)
    zeros16 = jnp.zeros((LANES,), jnp.int32)

    @functools.partial(plsc.parallel_loop, 0, BINS, unroll=8)
    def _(i):
        hist[pl.ds(i * LANES, LANES)] = zeros16

    cps = [None, None]
    cps[0] = pltpu.async_copy(
        loss_hbm.at[pl.ds(base_row, SUBR), :], bufs.at[0], sems.at[0])
    for s in range(NSUB):
        cur = s % 2
        if s + 1 < NSUB:
            nxt = (s + 1) % 2
            cps[nxt] = pltpu.async_copy(
                loss_hbm.at[pl.ds(base_row + (s + 1) * SUBR, SUBR), :],
                bufs.at[nxt], sems.at[nxt])
        cps[cur].wait()
        buf = bufs.at[cur]

        @functools.partial(plsc.parallel_loop, 0, SUB_V, unroll=8)
        def _(i):
            r = i >> 5
            c = i & 31
            scatter(buf[r, pl.ds(c * LANES, LANES)])

    @functools.partial(plsc.parallel_loop, 0, HOUT, unroll=1)
    def _(r):
        for j in range(8):
            rows = (r * 8 + j) * LANES + lanes
            acc = zeros16
            for l in range(LANES):
                acc = acc + plsc.load_gather(hist, [rows * LANES + l])
            obuf[r, pl.ds(j * LANES, LANES)] = acc


@functools.partial(
    pl.kernel,
    out_type=jax.ShapeDtypeStruct((HOUT * NW, 128), jnp.int32),
    mesh=_sc_mesh,
    compiler_params=pltpu.CompilerParams(needs_layout_passes=False),
    scratch_types=[
        pltpu.VMEM((2, SUBR, COLS), jnp.int32),
        pltpu.VMEM((BINS * LANES,), jnp.int32),
        pltpu.VMEM((HOUT, 128), jnp.int32),
        pltpu.SemaphoreType.DMA((2,)),
    ],
)
def _hist1_sc(loss_hbm, out_hbm, bufs, hist, obuf, sems):
    wid = lax.axis_index("s") * 2 + lax.axis_index("c")
    lanes = lax.iota(jnp.int32, LANES)
    ones16 = jnp.ones((LANES,), jnp.int32)

    def scatter(u):
        d = u >> SH1
        plsc.addupdate_scatter(hist, [d * LANES + lanes], ones16)

    _hist_tile(loss_hbm, bufs, hist, obuf, sems, wid * TROWS, scatter)
    pltpu.sync_copy(obuf, out_hbm.at[pl.ds(wid * HOUT, HOUT), :])


@functools.partial(
    pl.kernel,
    out_type=jax.ShapeDtypeStruct((HOUT * NW, 128), jnp.int32),
    mesh=_sc_mesh,
    compiler_params=pltpu.CompilerParams(needs_layout_passes=False),
    scratch_types=[
        pltpu.VMEM((2, SUBR, COLS), jnp.int32),
        pltpu.VMEM((BINS * LANES,), jnp.int32),
        pltpu.VMEM((HOUT, 128), jnp.int32),
        pltpu.VMEM((LANES,), jnp.int32),
        pltpu.SemaphoreType.DMA((2,)),
    ],
)
def _hist2_sc(loss_hbm, c1_hbm, out_hbm, bufs, hist, obuf, b1buf, sems):
    wid = lax.axis_index("s") * 2 + lax.axis_index("c")
    lanes = lax.iota(jnp.int32, LANES)
    ones16 = jnp.ones((LANES,), jnp.int32)

    pltpu.sync_copy(c1_hbm.at[0, pl.ds(0, LANES)], b1buf)
    b1v = plsc.load_gather(b1buf, [jnp.zeros((LANES,), jnp.int32)])

    def scatter(u):
        d1 = u >> SH1
        d2 = (u >> SH2) & 0xFFF
        plsc.addupdate_scatter(
            hist, [d2 * LANES + lanes], ones16, mask=d1 == b1v)

    _hist_tile(loss_hbm, bufs, hist, obuf, sems, wid * TROWS, scatter)
    pltpu.sync_copy(obuf, out_hbm.at[pl.ds(wid * HOUT, HOUT), :])


def _suffix_counts(h_ref):
    hw = h_ref[...].astype(jnp.float32)
    h = jnp.zeros((32, 128), jnp.float32)
    for w in range(NW):
        h = h + hw[w * 32:(w + 1) * 32, :]
    cs = lax.broadcasted_iota(jnp.int32, (128, 128), 0)
    ct = lax.broadcasted_iota(jnp.int32, (128, 128), 1)
    m_in = (cs >= ct).astype(jnp.float32)
    inrow = jnp.dot(h, m_in, preferred_element_type=jnp.float32)
    rs = lax.broadcasted_iota(jnp.int32, (32, 32), 0)
    rt = lax.broadcasted_iota(jnp.int32, (32, 32), 1)
    m_tail = (rt > rs).astype(jnp.float32)
    tail = jnp.sum(jnp.dot(m_tail, h, preferred_element_type=jnp.float32),
                   axis=1, keepdims=True)
    idx = (lax.broadcasted_iota(jnp.int32, (32, 128), 0) * 128
           + lax.broadcasted_iota(jnp.int32, (32, 128), 1))
    return inrow + tail, idx, h


def _pick1_body(h_ref, o_ref):
    C, idx, h = _suffix_counts(h_ref)
    b1 = jnp.max(jnp.where(C >= float(K), idx, -1))
    c_above = jnp.sum(jnp.where(idx > b1, h, 0.0))
    r1 = (float(K) - c_above).astype(jnp.int32)
    orow = lax.broadcasted_iota(jnp.int32, (8, 128), 0)
    ocol = lax.broadcasted_iota(jnp.int32, (8, 128), 1)
    flat = orow * 128 + ocol
    o_ref[...] = jnp.where(flat == 0, b1, jnp.where(flat == 1, r1, 0))


def _pick2_body(h_ref, c1_ref, o_ref):
    r1 = c1_ref[0, 1].astype(jnp.float32)
    b1 = c1_ref[0, 0]
    C, idx, _ = _suffix_counts(h_ref)
    b2 = jnp.max(jnp.where(C >= r1, idx, -1))
    edge = b1 * (1 << SH1) + b2 * (1 << SH2)
    orow = lax.broadcasted_iota(jnp.int32, (8, 128), 0)
    ocol = lax.broadcasted_iota(jnp.int32, (8, 128), 1)
    flat = orow * 128 + ocol
    o_ref[...] = jnp.where(flat == 0, edge, 0)


def _final_body(c2_ref, x_ref, o_ref):
    i = pl.program_id(0)
    edge = c2_ref[0, 0]
    u = x_ref[...]
    x = lax.bitcast_convert_type(u, jnp.float32)
    keep = u >= edge + (1 << SH2)
    s_part = jnp.sum(jnp.where(keep, x, 0.0))
    c_part = jnp.sum(keep.astype(jnp.float32))
    lane = lax.broadcasted_iota(jnp.int32, (1, 128), 1)
    part = jnp.where(lane == 0, s_part, jnp.where(lane == 1, c_part, 0.0))

    @pl.when(i == 0)
    def _():
        o_ref[...] = jnp.zeros_like(o_ref)

    o_ref[...] += part

    @pl.when(i == GRID_R - 1)
    def _():
        r = o_ref[...]
        s_tot = r[0, 0]
        c_tot = r[0, 1]
        edge_f = lax.bitcast_convert_type(edge, jnp.float32)
        res = (s_tot + (float(K) - c_tot) * edge_f) / float(K)
        o_ref[...] = jnp.where(lane == 2, res, r)


def kernel(logits, targets):
    x2d = logits.reshape(ROWS, COLS)
    t2d = targets.reshape(ROWS, COLS)

    bits = pl.pallas_call(
        _bce_body,
        grid=(GRID_R,),
        in_specs=[pl.BlockSpec((BLK_R, COLS), lambda i: (i, 0))] * 2,
        out_specs=pl.BlockSpec((BLK_R, COLS), lambda i: (i, 0)),
        out_shape=jax.ShapeDtypeStruct((ROWS, COLS), jnp.int32),
    )(x2d, t2d)

    h1 = _hist1_sc(bits)
    c1 = pl.pallas_call(
        _pick1_body,
        out_shape=jax.ShapeDtypeStruct((8, 128), jnp.int32),
    )(h1)

    h2 = _hist2_sc(bits, c1)
    c2 = pl.pallas_call(
        _pick2_body,
        out_shape=jax.ShapeDtypeStruct((8, 128), jnp.int32),
    )(h2, c1)

    outv = pl.pallas_call(
        _final_body,
        grid=(GRID_R,),
        in_specs=[
            pl.BlockSpec((8, 128), lambda i: (0, 0)),
            pl.BlockSpec((BLK_R, COLS), lambda i: (i, 0)),
        ],
        out_specs=pl.BlockSpec((1, 128), lambda i: (0, 0)),
        out_shape=jax.ShapeDtypeStruct((1, 128), jnp.float32),
    )(c2, bits)

    return outv[0, 2]


# speedup vs baseline: 68.6311x; 1.0819x over previous
import functools

import jax
import jax.numpy as jnp
from jax import lax
from jax.experimental import pallas as pl
from jax.experimental.pallas import tpu as pltpu
from jax.experimental.pallas import tpu_sc as plsc

N = 16 * 512 * 512
K = N // 4
ROWS, COLS = 8192, 512
GRID_R = 8
BLK_R = ROWS // GRID_R
NW = 32
TROWS = ROWS // NW
SUBR = 32
NSUB = TROWS // SUBR
SUB_V = SUBR * COLS // 16
BINS = 4096
HOUT = BINS // 128
SH1 = 19
SH2 = 7
LANES = 16


def _bce_body(x_ref, t_ref, o_ref):
    x = x_ref[...]
    t = t_ref[...]
    loss = jnp.maximum(x, 0.0) - x * t + jnp.log1p(jnp.exp(-jnp.abs(x)))
    o_ref[...] = lax.bitcast_convert_type(loss, jnp.int32)


_sc_mesh = plsc.VectorSubcoreMesh(core_axis_name="c", subcore_axis_name="s")


def _hist_tile(loss_hbm, bufs, hist, obuf, sems, base_row, scatter):
    lanes = lax.iota(jnp.int32, LANES)
    zeros16 = jnp.zeros((LANES,), jnp.int32)

    @functools.partial(plsc.parallel_loop, 0, BINS, unroll=8)
    def _(i):
        hist[pl.ds(i * LANES, LANES)] = zeros16

    cps = [None, None]
    cps[0] = pltpu.async_copy(
        loss_hbm.at[pl.ds(base_row, SUBR), :], bufs.at[0], sems.at[0])
    for s in range(NSUB):
        cur = s % 2
        if s + 1 < NSUB:
            nxt = (s + 1) % 2
            cps[nxt] = pltpu.async_copy(
                loss_hbm.at[pl.ds(base_row + (s + 1) * SUBR, SUBR), :],
                bufs.at[nxt], sems.at[nxt])
        cps[cur].wait()
        buf = bufs.at[cur]

        @functools.partial(plsc.parallel_loop, 0, SUB_V, unroll=8)
        def _(i):
            r = i >> 5
            c = i & 31
            scatter(buf[r, pl.ds(c * LANES, LANES)])

    @functools.partial(plsc.parallel_loop, 0, HOUT, unroll=1)
    def _(r):
        for j in range(8):
            rows = (r * 8 + j) * LANES + lanes
            acc = zeros16
            for l in range(LANES):
                acc = acc + plsc.load_gather(hist, [rows * LANES + l])
            obuf[r, pl.ds(j * LANES, LANES)] = acc


@functools.partial(
    pl.kernel,
    out_type=jax.ShapeDtypeStruct((HOUT * NW, 128), jnp.int32),
    mesh=_sc_mesh,
    compiler_params=pltpu.CompilerParams(needs_layout_passes=False),
    scratch_types=[
        pltpu.VMEM((2, SUBR, COLS), jnp.int32),
        pltpu.VMEM((BINS * LANES,), jnp.int32),
        pltpu.VMEM((HOUT, 128), jnp.int32),
        pltpu.SemaphoreType.DMA((2,)),
    ],
)
def _hist1_sc(loss_hbm, out_hbm, bufs, hist, obuf, sems):
    wid = lax.axis_index("s") * 2 + lax.axis_index("c")
    lanes = lax.iota(jnp.int32, LANES)
    ones16 = jnp.ones((LANES,), jnp.int32)

    def scatter(u):
        d = u >> SH1
        plsc.addupdate_scatter(hist, [d * LANES + lanes], ones16)

    _hist_tile(loss_hbm, bufs, hist, obuf, sems, wid * TROWS, scatter)
    pltpu.sync_copy(obuf, out_hbm.at[pl.ds(wid * HOUT, HOUT), :])


@functools.partial(
    pl.kernel,
    out_type=jax.ShapeDtypeStruct((HOUT * NW, 128), jnp.int32),
    mesh=_sc_mesh,
    compiler_params=pltpu.CompilerParams(needs_layout_passes=False),
    scratch_types=[
        pltpu.VMEM((2, SUBR, COLS), jnp.int32),
        pltpu.VMEM((BINS * LANES,), jnp.int32),
        pltpu.VMEM((HOUT, 128), jnp.int32),
        pltpu.VMEM((LANES,), jnp.int32),
        pltpu.SemaphoreType.DMA((2,)),
    ],
)
def _hist2_sc(loss_hbm, c1_hbm, out_hbm, bufs, hist, obuf, b1buf, sems):
    wid = lax.axis_index("s") * 2 + lax.axis_index("c")
    lanes = lax.iota(jnp.int32, LANES)
    ones16 = jnp.ones((LANES,), jnp.int32)

    pltpu.sync_copy(c1_hbm.at[0, pl.ds(0, LANES)], b1buf)
    b1v = plsc.load_gather(b1buf, [jnp.zeros((LANES,), jnp.int32)])

    def scatter(u):
        d1 = u >> SH1
        d2 = (u >> SH2) & 0xFFF
        plsc.addupdate_scatter(
            hist, [d2 * LANES + lanes], ones16, mask=d1 == b1v)

    _hist_tile(loss_hbm, bufs, hist, obuf, sems, wid * TROWS, scatter)
    pltpu.sync_copy(obuf, out_hbm.at[pl.ds(wid * HOUT, HOUT), :])


def _suffix_counts(h_ref):
    hw = h_ref[...].astype(jnp.float32)
    h = jnp.zeros((32, 128), jnp.float32)
    for w in range(NW):
        h = h + hw[w * 32:(w + 1) * 32, :]
    cs = lax.broadcasted_iota(jnp.int32, (128, 128), 0)
    ct = lax.broadcasted_iota(jnp.int32, (128, 128), 1)
    m_in = (cs >= ct).astype(jnp.float32)
    inrow = jnp.dot(h, m_in, preferred_element_type=jnp.float32)
    rs = lax.broadcasted_iota(jnp.int32, (32, 32), 0)
    rt = lax.broadcasted_iota(jnp.int32, (32, 32), 1)
    m_tail = (rt > rs).astype(jnp.float32)
    tail = jnp.sum(jnp.dot(m_tail, h, preferred_element_type=jnp.float32),
                   axis=1, keepdims=True)
    idx = (lax.broadcasted_iota(jnp.int32, (32, 128), 0) * 128
           + lax.broadcasted_iota(jnp.int32, (32, 128), 1))
    return inrow + tail, idx, h


def _pick1_body(h_ref, o_ref):
    C, idx, h = _suffix_counts(h_ref)
    b1 = jnp.max(jnp.where(C >= float(K), idx, -1))
    c_above = jnp.sum(jnp.where(idx > b1, h, 0.0))
    r1 = (float(K) - c_above).astype(jnp.int32)
    orow = lax.broadcasted_iota(jnp.int32, (8, 128), 0)
    ocol = lax.broadcasted_iota(jnp.int32, (8, 128), 1)
    flat = orow * 128 + ocol
    o_ref[...] = jnp.where(flat == 0, b1, jnp.where(flat == 1, r1, 0))


def _final_body(h2_ref, c1_ref, x_ref, o_ref, edge_ref):
    i = pl.program_id(0)

    @pl.when(i == 0)
    def _():
        r1 = c1_ref[0, 1].astype(jnp.float32)
        b1 = c1_ref[0, 0]
        C, idx, _ = _suffix_counts(h2_ref)
        b2 = jnp.max(jnp.where(C >= r1, idx, -1))
        edge_ref[0] = b1 * (1 << SH1) + b2 * (1 << SH2)

    edge = edge_ref[0]
    u = x_ref[...]
    x = lax.bitcast_convert_type(u, jnp.float32)
    keep = u >= edge + (1 << SH2)
    s_part = jnp.sum(jnp.where(keep, x, 0.0))
    c_part = jnp.sum(keep.astype(jnp.float32))
    lane = lax.broadcasted_iota(jnp.int32, (1, 128), 1)
    part = jnp.where(lane == 0, s_part, jnp.where(lane == 1, c_part, 0.0))

    @pl.when(i == 0)
    def _():
        o_ref[...] = jnp.zeros_like(o_ref)

    o_ref[...] += part

    @pl.when(i == GRID_R - 1)
    def _():
        r = o_ref[...]
        s_tot = r[0, 0]
        c_tot = r[0, 1]
        edge_f = lax.bitcast_convert_type(edge, jnp.float32)
        res = (s_tot + (float(K) - c_tot) * edge_f) / float(K)
        o_ref[...] = jnp.where(lane == 2, res, r)


def kernel(logits, targets):
    x2d = logits.reshape(ROWS, COLS)
    t2d = targets.reshape(ROWS, COLS)

    bits = pl.pallas_call(
        _bce_body,
        grid=(GRID_R,),
        in_specs=[pl.BlockSpec((BLK_R, COLS), lambda i: (i, 0))] * 2,
        out_specs=pl.BlockSpec((BLK_R, COLS), lambda i: (i, 0)),
        out_shape=jax.ShapeDtypeStruct((ROWS, COLS), jnp.int32),
    )(x2d, t2d)

    h1 = _hist1_sc(bits)
    c1 = pl.pallas_call(
        _pick1_body,
        out_shape=jax.ShapeDtypeStruct((8, 128), jnp.int32),
    )(h1)

    h2 = _hist2_sc(bits, c1)

    outv = pl.pallas_call(
        _final_body,
        grid=(GRID_R,),
        in_specs=[
            pl.BlockSpec((HOUT * NW, 128), lambda i: (0, 0)),
            pl.BlockSpec((8, 128), lambda i: (0, 0)),
            pl.BlockSpec((BLK_R, COLS), lambda i: (i, 0)),
        ],
        out_specs=pl.BlockSpec((1, 128), lambda i: (0, 0)),
        out_shape=jax.ShapeDtypeStruct((1, 128), jnp.float32),
        scratch_shapes=[pltpu.SMEM((1,), jnp.int32)],
    )(h2, c1, bits)

    return outv[0, 2]


# speedup vs baseline: 68.6788x; 1.0007x over previous
import functools

import jax
import jax.numpy as jnp
from jax import lax
from jax.experimental import pallas as pl
from jax.experimental.pallas import tpu as pltpu
from jax.experimental.pallas import tpu_sc as plsc

N = 16 * 512 * 512
K = N // 4
ROWS, COLS = 8192, 512
GRID_R = 8
BLK_R = ROWS // GRID_R
NW = 32
TROWS = ROWS // NW
SUBR = 32
NSUB = TROWS // SUBR
SUB_V = SUBR * COLS // 16
BINS = 4096
HOUT = BINS // 128
SH1 = 19
SH2 = 7
LANES = 16


def _bce_body(x_ref, t_ref, o_ref):
    x = x_ref[...]
    t = t_ref[...]
    loss = jnp.maximum(x, 0.0) - x * t + jnp.log1p(jnp.exp(-jnp.abs(x)))
    o_ref[...] = lax.bitcast_convert_type(loss, jnp.int32)


_sc_mesh = plsc.VectorSubcoreMesh(core_axis_name="c", subcore_axis_name="s")


def _hist_tile(loss_hbm, bufs, hist, obuf, sems, base_row, scatter):
    lanes = lax.iota(jnp.int32, LANES)
    zeros16 = jnp.zeros((LANES,), jnp.int32)

    @functools.partial(plsc.parallel_loop, 0, BINS, unroll=8)
    def _(i):
        hist[pl.ds(i * LANES, LANES)] = zeros16

    cps = [None, None]
    cps[0] = pltpu.async_copy(
        loss_hbm.at[pl.ds(base_row, SUBR), :], bufs.at[0], sems.at[0])
    for s in range(NSUB):
        cur = s % 2
        if s + 1 < NSUB:
            nxt = (s + 1) % 2
            cps[nxt] = pltpu.async_copy(
                loss_hbm.at[pl.ds(base_row + (s + 1) * SUBR, SUBR), :],
                bufs.at[nxt], sems.at[nxt])
        cps[cur].wait()
        buf = bufs.at[cur]

        @functools.partial(plsc.parallel_loop, 0, SUB_V, unroll=16)
        def _(i):
            r = i >> 5
            c = i & 31
            scatter(buf[r, pl.ds(c * LANES, LANES)])

    @functools.partial(plsc.parallel_loop, 0, HOUT, unroll=1)
    def _(r):
        for j in range(8):
            rows = (r * 8 + j) * LANES + lanes
            acc = zeros16
            for l in range(LANES):
                acc = acc + plsc.load_gather(hist, [rows * LANES + l])
            obuf[r, pl.ds(j * LANES, LANES)] = acc


@functools.partial(
    pl.kernel,
    out_type=jax.ShapeDtypeStruct((HOUT * NW, 128), jnp.int32),
    mesh=_sc_mesh,
    compiler_params=pltpu.CompilerParams(needs_layout_passes=False),
    scratch_types=[
        pltpu.VMEM((2, SUBR, COLS), jnp.int32),
        pltpu.VMEM((BINS * LANES,), jnp.int32),
        pltpu.VMEM((HOUT, 128), jnp.int32),
        pltpu.SemaphoreType.DMA((2,)),
    ],
)
def _hist1_sc(loss_hbm, out_hbm, bufs, hist, obuf, sems):
    wid = lax.axis_index("s") * 2 + lax.axis_index("c")
    lanes = lax.iota(jnp.int32, LANES)
    ones16 = jnp.ones((LANES,), jnp.int32)

    def scatter(u):
        d = u >> SH1
        plsc.addupdate_scatter(hist, [d * LANES + lanes], ones16)

    _hist_tile(loss_hbm, bufs, hist, obuf, sems, wid * TROWS, scatter)
    pltpu.sync_copy(obuf, out_hbm.at[pl.ds(wid * HOUT, HOUT), :])


@functools.partial(
    pl.kernel,
    out_type=jax.ShapeDtypeStruct((HOUT * NW, 128), jnp.int32),
    mesh=_sc_mesh,
    compiler_params=pltpu.CompilerParams(needs_layout_passes=False),
    scratch_types=[
        pltpu.VMEM((2, SUBR, COLS), jnp.int32),
        pltpu.VMEM((BINS * LANES,), jnp.int32),
        pltpu.VMEM((HOUT, 128), jnp.int32),
        pltpu.VMEM((LANES,), jnp.int32),
        pltpu.SemaphoreType.DMA((2,)),
    ],
)
def _hist2_sc(loss_hbm, c1_hbm, out_hbm, bufs, hist, obuf, b1buf, sems):
    wid = lax.axis_index("s") * 2 + lax.axis_index("c")
    lanes = lax.iota(jnp.int32, LANES)
    ones16 = jnp.ones((LANES,), jnp.int32)

    pltpu.sync_copy(c1_hbm.at[0, pl.ds(0, LANES)], b1buf)
    b1v = plsc.load_gather(b1buf, [jnp.zeros((LANES,), jnp.int32)])

    def scatter(u):
        d1 = u >> SH1
        d2 = (u >> SH2) & 0xFFF
        plsc.addupdate_scatter(
            hist, [d2 * LANES + lanes], ones16, mask=d1 == b1v)

    _hist_tile(loss_hbm, bufs, hist, obuf, sems, wid * TROWS, scatter)
    pltpu.sync_copy(obuf, out_hbm.at[pl.ds(wid * HOUT, HOUT), :])


def _suffix_counts(h_ref):
    hw = h_ref[...].astype(jnp.float32)
    h = jnp.zeros((32, 128), jnp.float32)
    for w in range(NW):
        h = h + hw[w * 32:(w + 1) * 32, :]
    cs = lax.broadcasted_iota(jnp.int32, (128, 128), 0)
    ct = lax.broadcasted_iota(jnp.int32, (128, 128), 1)
    m_in = (cs >= ct).astype(jnp.float32)
    inrow = jnp.dot(h, m_in, preferred_element_type=jnp.float32)
    rs = lax.broadcasted_iota(jnp.int32, (32, 32), 0)
    rt = lax.broadcasted_iota(jnp.int32, (32, 32), 1)
    m_tail = (rt > rs).astype(jnp.float32)
    tail = jnp.sum(jnp.dot(m_tail, h, preferred_element_type=jnp.float32),
                   axis=1, keepdims=True)
    idx = (lax.broadcasted_iota(jnp.int32, (32, 128), 0) * 128
           + lax.broadcasted_iota(jnp.int32, (32, 128), 1))
    return inrow + tail, idx, h


def _pick1_body(h_ref, o_ref):
    C, idx, h = _suffix_counts(h_ref)
    b1 = jnp.max(jnp.where(C >= float(K), idx, -1))
    c_above = jnp.sum(jnp.where(idx > b1, h, 0.0))
    r1 = (float(K) - c_above).astype(jnp.int32)
    orow = lax.broadcasted_iota(jnp.int32, (8, 128), 0)
    ocol = lax.broadcasted_iota(jnp.int32, (8, 128), 1)
    flat = orow * 128 + ocol
    o_ref[...] = jnp.where(flat == 0, b1, jnp.where(flat == 1, r1, 0))


def _final_body(h2_ref, c1_ref, x_ref, o_ref, edge_ref):
    i = pl.program_id(0)

    @pl.when(i == 0)
    def _():
        r1 = c1_ref[0, 1].astype(jnp.float32)
        b1 = c1_ref[0, 0]
        C, idx, h2m = _suffix_counts(h2_ref)
        b2 = jnp.max(jnp.where(C >= r1, idx, -1))
        edge_ref[0] = b1 * (1 << SH1) + b2 * (1 << SH2)
        c_up = (float(K) - r1) + jnp.sum(jnp.where(idx > b2, h2m, 0.0))
        edge_ref[1] = c_up.astype(jnp.int32)

    edge = edge_ref[0]
    u = x_ref[...]
    x = lax.bitcast_convert_type(u, jnp.float32)
    keep = u >= edge + (1 << SH2)
    s_part = jnp.sum(jnp.where(keep, x, 0.0))
    lane = lax.broadcasted_iota(jnp.int32, (1, 128), 1)
    part = jnp.where(lane == 0, s_part, 0.0)

    @pl.when(i == 0)
    def _():
        o_ref[...] = jnp.zeros_like(o_ref)

    o_ref[...] += part

    @pl.when(i == GRID_R - 1)
    def _():
        r = o_ref[...]
        s_tot = r[0, 0]
        c_tot = edge_ref[1].astype(jnp.float32)
        edge_f = lax.bitcast_convert_type(edge, jnp.float32)
        res = (s_tot + (float(K) - c_tot) * edge_f) / float(K)
        o_ref[...] = jnp.where(lane == 2, res, r)


def kernel(logits, targets):
    x2d = logits.reshape(ROWS, COLS)
    t2d = targets.reshape(ROWS, COLS)

    bits = pl.pallas_call(
        _bce_body,
        grid=(GRID_R,),
        in_specs=[pl.BlockSpec((BLK_R, COLS), lambda i: (i, 0))] * 2,
        out_specs=pl.BlockSpec((BLK_R, COLS), lambda i: (i, 0)),
        out_shape=jax.ShapeDtypeStruct((ROWS, COLS), jnp.int32),
    )(x2d, t2d)

    h1 = _hist1_sc(bits)
    c1 = pl.pallas_call(
        _pick1_body,
        out_shape=jax.ShapeDtypeStruct((8, 128), jnp.int32),
    )(h1)

    h2 = _hist2_sc(bits, c1)

    outv = pl.pallas_call(
        _final_body,
        grid=(GRID_R,),
        in_specs=[
            pl.BlockSpec((HOUT * NW, 128), lambda i: (0, 0)),
            pl.BlockSpec((8, 128), lambda i: (0, 0)),
            pl.BlockSpec((BLK_R, COLS), lambda i: (i, 0)),
        ],
        out_specs=pl.BlockSpec((1, 128), lambda i: (0, 0)),
        out_shape=jax.ShapeDtypeStruct((1, 128), jnp.float32),
        scratch_shapes=[pltpu.SMEM((2,), jnp.int32)],
    )(h2, c1, bits)

    return outv[0, 2]
